# Initial kernel scaffold; baseline (speedup 1.0000x reference)
import jax
import jax.numpy as jnp
from jax.experimental import pallas as pl


def kernel(x, edge_index, w_comp, bases, h_bias):
    raise NotImplementedError("write your pallas kernel here")


# speedup vs baseline: 2.5562x; 2.5562x over previous
import functools

import jax
import jax.numpy as jnp
from jax import lax
from jax.experimental import pallas as pl
from jax.experimental.pallas import tpu as pltpu
from jax.experimental.pallas import tpu_sc as plsc

N_NODES = 10000
IN_FEAT = 128
OUT_FEAT = 128
NUM_RELS = 8
NUM_BASES = 4
N_EDGES = 320000
ER = N_EDGES // NUM_RELS

NC = 2
NS = 16
LANES = 16
CHUNK = 128
CHUNKS_PER_TILE = 20
ER_PAD = NS * CHUNKS_PER_TILE * CHUNK
RELS_PER_CORE = NUM_RELS // NC

N_PAD = 10240
ROWS_PER_TILE = N_PAD // NS
FLUSH_CHUNKS = ROWS_PER_TILE // CHUNK

TC_BLK = 2000


_sc_mesh = plsc.VectorSubcoreMesh(core_axis_name="c", subcore_axis_name="s")


@functools.partial(
    pl.kernel,
    out_type=(
        jax.ShapeDtypeStruct((NUM_RELS, N_PAD, IN_FEAT), jnp.float32),
        jax.ShapeDtypeStruct((NUM_RELS, N_PAD, IN_FEAT), jnp.float32),
    ),
    mesh=_sc_mesh,
    scratch_types=[
        pltpu.VMEM((CHUNK,), jnp.int32),
        pltpu.VMEM((CHUNK,), jnp.int32),
        pltpu.VMEM((CHUNK, IN_FEAT), jnp.float32),
        pltpu.VMEM_SHARED((N_PAD, IN_FEAT), jnp.float32),
    ],
)
def _sc_segment_sums(x_hbm, src_hbm, dst_hbm, z128_hbm, ones_hbm,
                     agg_out, deg_out,
                     src_t, dst_t, rows_v, acc):
    c = lax.axis_index("c")
    s = lax.axis_index("s")
    row0 = s * ROWS_PER_TILE

    def zero_own_rows():
        pltpu.sync_copy(z128_hbm, rows_v)
        for j in range(FLUSH_CHUNKS):
            pltpu.sync_copy(rows_v, acc.at[pl.ds(row0 + j * CHUNK, CHUNK)])

    def flush_own_rows(out_ref, r):
        for j in range(FLUSH_CHUNKS):
            rows = pl.ds(row0 + j * CHUNK, CHUNK)
            pltpu.sync_copy(acc.at[rows], rows_v)
            pltpu.sync_copy(rows_v, out_ref.at[r, rows])

    for rr in range(RELS_PER_CORE):
        r = c * RELS_PER_CORE + rr
        zero_own_rows()
        plsc.subcore_barrier()

        @pl.loop(0, CHUNKS_PER_TILE)
        def _(k):
            pltpu.sync_copy(src_hbm.at[r, s, k], src_t)
            pltpu.sync_copy(dst_hbm.at[r, s, k], dst_t)
            pltpu.sync_copy(x_hbm.at[src_t], rows_v)
            pltpu.sync_copy(rows_v, acc.at[dst_t], add=True)

        plsc.subcore_barrier()
        flush_own_rows(agg_out, r)
        zero_own_rows()
        plsc.subcore_barrier()

        pltpu.sync_copy(ones_hbm, rows_v)

        @pl.loop(0, CHUNKS_PER_TILE)
        def _(k):
            pltpu.sync_copy(dst_hbm.at[r, s, k], dst_t)
            pltpu.sync_copy(rows_v, acc.at[dst_t], add=True)

        plsc.subcore_barrier()
        flush_own_rows(deg_out, r)


def _tc_body(wc_ref, bases_ref, bias_ref, agg_ref, deg_ref, out_ref):
    acc = jnp.zeros((TC_BLK, OUT_FEAT), jnp.float32)
    for r in range(NUM_RELS):
        w_r = jnp.zeros((IN_FEAT, OUT_FEAT), jnp.float32)
        for b in range(NUM_BASES):
            w_r = w_r + wc_ref[r, b] * bases_ref[b]
        d = jnp.maximum(deg_ref[r, :, 0:1], 1.0)
        a = agg_ref[r] / d
        acc = acc + jnp.dot(a, w_r, preferred_element_type=jnp.float32,
                            precision=lax.Precision.HIGHEST)
    out_ref[...] = acc + bias_ref[...]


def _tc_combine(w_comp, bases, h_bias2d, agg, deg):
    return pl.pallas_call(
        _tc_body,
        grid=(N_NODES // TC_BLK,),
        in_specs=[
            pl.BlockSpec((NUM_RELS, NUM_BASES), lambda i: (0, 0),
                         memory_space=pltpu.SMEM),
            pl.BlockSpec((NUM_BASES, IN_FEAT, OUT_FEAT), lambda i: (0, 0, 0)),
            pl.BlockSpec((1, OUT_FEAT), lambda i: (0, 0)),
            pl.BlockSpec((NUM_RELS, TC_BLK, IN_FEAT), lambda i: (0, i, 0)),
            pl.BlockSpec((NUM_RELS, TC_BLK, IN_FEAT), lambda i: (0, i, 0)),
        ],
        out_specs=pl.BlockSpec((TC_BLK, OUT_FEAT), lambda i: (i, 0)),
        out_shape=jax.ShapeDtypeStruct((N_NODES, OUT_FEAT), jnp.float32),
    )(w_comp, bases, h_bias2d, agg, deg)


def kernel(x, edge_index, w_comp, bases, h_bias):
    src = edge_index[0].astype(jnp.int32).reshape(NUM_RELS, ER)
    dst = edge_index[1].astype(jnp.int32).reshape(NUM_RELS, ER)
    pad = ER_PAD - ER
    src_p = jnp.pad(src, ((0, 0), (0, pad))).reshape(
        NUM_RELS, NS, CHUNKS_PER_TILE, CHUNK)
    dst_p = jnp.pad(dst, ((0, 0), (0, pad)), constant_values=N_NODES).reshape(
        NUM_RELS, NS, CHUNKS_PER_TILE, CHUNK)
    z128 = jnp.zeros((CHUNK, IN_FEAT), jnp.float32)
    ones128 = jnp.ones((CHUNK, IN_FEAT), jnp.float32)
    agg, deg = _sc_segment_sums(x, src_p, dst_p, z128, ones128)
    return _tc_combine(w_comp, bases, h_bias.reshape(1, OUT_FEAT), agg, deg)


# speedup vs baseline: 2.9539x; 1.1556x over previous
import functools

import jax
import jax.numpy as jnp
from jax import lax
from jax.experimental import pallas as pl
from jax.experimental.pallas import tpu as pltpu
from jax.experimental.pallas import tpu_sc as plsc

N_NODES = 10000
IN_FEAT = 128
OUT_FEAT = 128
NUM_RELS = 8
NUM_BASES = 4
N_EDGES = 320000
ER = N_EDGES // NUM_RELS

NC = 2
NS = 16
LANES = 16
CHUNK = 128
CHUNKS_PER_TILE = 20
ER_PAD = NS * CHUNKS_PER_TILE * CHUNK
RELS_PER_CORE = NUM_RELS // NC

N_PAD = 10240
ROWS_PER_TILE = N_PAD // NS
FLUSH_CHUNKS = ROWS_PER_TILE // CHUNK

TC_BLK = 2000


_sc_mesh = plsc.VectorSubcoreMesh(core_axis_name="c", subcore_axis_name="s")


@functools.partial(
    pl.kernel,
    out_type=(
        jax.ShapeDtypeStruct((NUM_RELS, N_PAD, IN_FEAT), jnp.float32),
        jax.ShapeDtypeStruct((NUM_RELS, N_PAD, IN_FEAT), jnp.float32),
    ),
    mesh=_sc_mesh,
    scratch_types=[
        pltpu.VMEM((CHUNKS_PER_TILE, CHUNK), jnp.int32),
        pltpu.VMEM((CHUNKS_PER_TILE, CHUNK), jnp.int32),
        pltpu.VMEM((CHUNK, IN_FEAT), jnp.float32),
        pltpu.VMEM((CHUNK, IN_FEAT), jnp.float32),
        pltpu.VMEM_SHARED((N_PAD, IN_FEAT), jnp.float32),
        pltpu.SemaphoreType.DMA,
    ],
)
def _sc_segment_sums(x_hbm, src_hbm, dst_hbm, z128_hbm, ones_hbm,
                     agg_out, deg_out,
                     src_t, dst_t, rows_a, rows_b, acc, gsem):
    c = lax.axis_index("c")
    s = lax.axis_index("s")
    row0 = s * ROWS_PER_TILE

    def zero_own_rows():
        pltpu.sync_copy(z128_hbm, rows_a)
        for j in range(FLUSH_CHUNKS):
            pltpu.sync_copy(rows_a, acc.at[pl.ds(row0 + j * CHUNK, CHUNK)])

    def flush_own_rows(out_ref, r):
        for j in range(FLUSH_CHUNKS):
            rows = pl.ds(row0 + j * CHUNK, CHUNK)
            pltpu.sync_copy(acc.at[rows], rows_a)
            pltpu.sync_copy(rows_a, out_ref.at[r, rows])

    for rr in range(RELS_PER_CORE):
        r = c * RELS_PER_CORE + rr
        zero_own_rows()
        plsc.subcore_barrier()

        pltpu.sync_copy(src_hbm.at[r, s], src_t)
        pltpu.sync_copy(dst_hbm.at[r, s], dst_t)
        cp = pltpu.async_copy(x_hbm.at[src_t.at[0]], rows_a, gsem)
        for k in range(CHUNKS_PER_TILE):
            buf = rows_a if k % 2 == 0 else rows_b
            nxt = rows_b if k % 2 == 0 else rows_a
            cp.wait()
            if k + 1 < CHUNKS_PER_TILE:
                cp = pltpu.async_copy(x_hbm.at[src_t.at[k + 1]], nxt, gsem)
            pltpu.sync_copy(buf, acc.at[dst_t.at[k]], add=True)

        plsc.subcore_barrier()
        flush_own_rows(agg_out, r)
        zero_own_rows()
        plsc.subcore_barrier()

        pltpu.sync_copy(ones_hbm, rows_a)
        for k in range(CHUNKS_PER_TILE):
            pltpu.sync_copy(rows_a, acc.at[dst_t.at[k]], add=True)

        plsc.subcore_barrier()
        flush_own_rows(deg_out, r)


def _tc_body(wc_ref, bases_ref, bias_ref, agg_ref, deg_ref, out_ref):
    acc = jnp.zeros((TC_BLK, OUT_FEAT), jnp.float32)
    for r in range(NUM_RELS):
        w_r = jnp.zeros((IN_FEAT, OUT_FEAT), jnp.float32)
        for b in range(NUM_BASES):
            w_r = w_r + wc_ref[r, b] * bases_ref[b]
        d = jnp.maximum(deg_ref[r, :, 0:1], 1.0)
        a = agg_ref[r] / d
        acc = acc + jnp.dot(a, w_r, preferred_element_type=jnp.float32,
                            precision=lax.Precision.HIGHEST)
    out_ref[...] = acc + bias_ref[...]


def _tc_combine(w_comp, bases, h_bias2d, agg, deg):
    return pl.pallas_call(
        _tc_body,
        grid=(N_NODES // TC_BLK,),
        in_specs=[
            pl.BlockSpec((NUM_RELS, NUM_BASES), lambda i: (0, 0),
                         memory_space=pltpu.SMEM),
            pl.BlockSpec((NUM_BASES, IN_FEAT, OUT_FEAT), lambda i: (0, 0, 0)),
            pl.BlockSpec((1, OUT_FEAT), lambda i: (0, 0)),
            pl.BlockSpec((NUM_RELS, TC_BLK, IN_FEAT), lambda i: (0, i, 0)),
            pl.BlockSpec((NUM_RELS, TC_BLK, IN_FEAT), lambda i: (0, i, 0)),
        ],
        out_specs=pl.BlockSpec((TC_BLK, OUT_FEAT), lambda i: (i, 0)),
        out_shape=jax.ShapeDtypeStruct((N_NODES, OUT_FEAT), jnp.float32),
    )(w_comp, bases, h_bias2d, agg, deg)


def kernel(x, edge_index, w_comp, bases, h_bias):
    src = edge_index[0].astype(jnp.int32).reshape(NUM_RELS, ER)
    dst = edge_index[1].astype(jnp.int32).reshape(NUM_RELS, ER)
    pad = ER_PAD - ER
    src_p = jnp.pad(src, ((0, 0), (0, pad))).reshape(
        NUM_RELS, NS, CHUNKS_PER_TILE, CHUNK)
    dst_p = jnp.pad(dst, ((0, 0), (0, pad)), constant_values=N_NODES).reshape(
        NUM_RELS, NS, CHUNKS_PER_TILE, CHUNK)
    z128 = jnp.zeros((CHUNK, IN_FEAT), jnp.float32)
    ones128 = jnp.ones((CHUNK, IN_FEAT), jnp.float32)
    agg, deg = _sc_segment_sums(x, src_p, dst_p, z128, ones128)
    return _tc_combine(w_comp, bases, h_bias.reshape(1, OUT_FEAT), agg, deg)


# speedup vs baseline: 3.4606x; 1.1715x over previous
import functools

import jax
import jax.numpy as jnp
from jax import lax
from jax.experimental import pallas as pl
from jax.experimental.pallas import tpu as pltpu
from jax.experimental.pallas import tpu_sc as plsc

N_NODES = 10000
IN_FEAT = 128
OUT_FEAT = 128
NUM_RELS = 8
NUM_BASES = 4
N_EDGES = 320000
ER = N_EDGES // NUM_RELS

NC = 2
NS = 16
CHUNK = 128
CHUNKS_PER_TILE = 20
ER_PAD = NS * CHUNKS_PER_TILE * CHUNK
RELS_PER_CORE = NUM_RELS // NC

N_PAD = 10240
ROWS_PER_TILE = N_PAD // NS
FLUSH_CHUNKS = ROWS_PER_TILE // CHUNK
HROWS = N_PAD // CHUNK
NVREG = ER_PAD // NS // 16

TC_BLK = 2048


_sc_mesh = plsc.VectorSubcoreMesh(core_axis_name="c", subcore_axis_name="s")


@functools.partial(
    pl.kernel,
    out_type=(
        jax.ShapeDtypeStruct((NUM_RELS, N_PAD, IN_FEAT), jnp.float32),
        jax.ShapeDtypeStruct((NUM_RELS, NS, N_PAD), jnp.float32),
    ),
    mesh=_sc_mesh,
    compiler_params=pltpu.CompilerParams(needs_layout_passes=False),
    scratch_types=[
        pltpu.VMEM((CHUNKS_PER_TILE, CHUNK), jnp.int32),
        pltpu.VMEM((CHUNKS_PER_TILE, CHUNK), jnp.int32),
        pltpu.VMEM((CHUNK, IN_FEAT), jnp.float32),
        pltpu.VMEM((CHUNK, IN_FEAT), jnp.float32),
        pltpu.VMEM((N_PAD,), jnp.float32),
        pltpu.VMEM_SHARED((N_PAD, IN_FEAT), jnp.float32),
        pltpu.SemaphoreType.DMA,
    ],
)
def _sc_segment_sums(x_hbm, src_hbm, dst_hbm, z128_hbm, z1d_hbm,
                     agg_out, deg_out,
                     src_t, dst_t, rows_a, rows_b, deg_hist, acc, gsem):
    c = lax.axis_index("c")
    s = lax.axis_index("s")
    row0 = s * ROWS_PER_TILE

    def zero_own_rows():
        pltpu.sync_copy(z128_hbm, rows_a)
        for j in range(FLUSH_CHUNKS):
            pltpu.sync_copy(rows_a, acc.at[pl.ds(row0 + j * CHUNK, CHUNK)])

    def flush_own_rows(out_ref, r):
        for j in range(FLUSH_CHUNKS):
            rows = pl.ds(row0 + j * CHUNK, CHUNK)
            pltpu.sync_copy(acc.at[rows], rows_a)
            pltpu.sync_copy(rows_a, out_ref.at[r, rows])

    iota16 = lax.iota(jnp.int32, 16)
    ones16 = jnp.ones((16,), jnp.float32)

    def hist_body(t, carry):
        v = dst_t[lax.shift_right_logical(t, 3),
                  pl.ds(jnp.bitwise_and(t, 7) * 16, 16)]
        plsc.addupdate_scatter(deg_hist, [v], ones16)
        return carry

    for rr in range(RELS_PER_CORE):
        r = c * RELS_PER_CORE + rr
        zero_own_rows()
        pltpu.sync_copy(z1d_hbm, deg_hist)
        plsc.subcore_barrier()

        pltpu.sync_copy(src_hbm.at[r, s], src_t)
        pltpu.sync_copy(dst_hbm.at[r, s], dst_t)
        cp = pltpu.async_copy(x_hbm.at[src_t.at[0]], rows_a, gsem)
        for k in range(CHUNKS_PER_TILE):
            buf = rows_a if k % 2 == 0 else rows_b
            nxt = rows_b if k % 2 == 0 else rows_a
            cp.wait()
            if k + 1 < CHUNKS_PER_TILE:
                cp = pltpu.async_copy(x_hbm.at[src_t.at[k + 1]], nxt, gsem)
            pltpu.sync_copy(buf, acc.at[dst_t.at[k]], add=True)

        lax.fori_loop(0, NVREG, hist_body, 0)
        pltpu.sync_copy(deg_hist, deg_out.at[r, s])

        plsc.subcore_barrier()
        flush_own_rows(agg_out, r)


def _tc_body(wc_ref, bases_ref, bias_ref, agg_ref, deg_ref, out_ref):
    acc = jnp.zeros((TC_BLK, OUT_FEAT), jnp.float32)
    for r in range(NUM_RELS):
        w_r = jnp.zeros((IN_FEAT, OUT_FEAT), jnp.float32)
        for b in range(NUM_BASES):
            w_r = w_r + wc_ref[r, b] * bases_ref[b]
        dsum = jnp.sum(deg_ref[r], axis=0, keepdims=True)
        d = jnp.maximum(jnp.transpose(dsum), 1.0)
        a = agg_ref[r] / d
        acc = acc + jnp.dot(a, w_r, preferred_element_type=jnp.float32,
                            precision=lax.Precision.HIGHEST)
    out_ref[...] = acc + bias_ref[...]


def _tc_combine(w_comp, bases, h_bias2d, agg, deg):
    return pl.pallas_call(
        _tc_body,
        grid=(N_PAD // TC_BLK,),
        in_specs=[
            pl.BlockSpec((NUM_RELS, NUM_BASES), lambda i: (0, 0),
                         memory_space=pltpu.SMEM),
            pl.BlockSpec((NUM_BASES, IN_FEAT, OUT_FEAT), lambda i: (0, 0, 0)),
            pl.BlockSpec((1, OUT_FEAT), lambda i: (0, 0)),
            pl.BlockSpec((NUM_RELS, TC_BLK, IN_FEAT), lambda i: (0, i, 0)),
            pl.BlockSpec((NUM_RELS, NS, TC_BLK), lambda i: (0, 0, i)),
        ],
        out_specs=pl.BlockSpec((TC_BLK, OUT_FEAT), lambda i: (i, 0)),
        out_shape=jax.ShapeDtypeStruct((N_PAD, OUT_FEAT), jnp.float32),
    )(w_comp, bases, h_bias2d, agg, deg)


def kernel(x, edge_index, w_comp, bases, h_bias):
    src = edge_index[0].astype(jnp.int32).reshape(NUM_RELS, ER)
    dst = edge_index[1].astype(jnp.int32).reshape(NUM_RELS, ER)
    pad = ER_PAD - ER
    src_p = jnp.pad(src, ((0, 0), (0, pad))).reshape(
        NUM_RELS, NS, CHUNKS_PER_TILE, CHUNK)
    dst_p = jnp.pad(dst, ((0, 0), (0, pad)), constant_values=N_NODES).reshape(
        NUM_RELS, NS, CHUNKS_PER_TILE, CHUNK)
    z128 = jnp.zeros((CHUNK, IN_FEAT), jnp.float32)
    z1d = jnp.zeros((N_PAD,), jnp.float32)
    agg, deg = _sc_segment_sums(x, src_p, dst_p, z128, z1d)
    out = _tc_combine(w_comp, bases, h_bias.reshape(1, OUT_FEAT), agg, deg)
    return out[:N_NODES]


# speedup vs baseline: 3.4961x; 1.0102x over previous
import functools

import jax
import jax.numpy as jnp
from jax import lax
from jax.experimental import pallas as pl
from jax.experimental.pallas import tpu as pltpu
from jax.experimental.pallas import tpu_sc as plsc

N_NODES = 10000
IN_FEAT = 128
OUT_FEAT = 128
NUM_RELS = 8
NUM_BASES = 4
N_EDGES = 320000
ER = N_EDGES // NUM_RELS

NC = 2
NS = 16
CHUNK = 128
CHUNKS_PER_TILE = 20
ER_PAD = NS * CHUNKS_PER_TILE * CHUNK
RELS_PER_CORE = NUM_RELS // NC

N_PAD = 10240
ROWS_PER_TILE = N_PAD // NS
FLUSH_CHUNKS = ROWS_PER_TILE // CHUNK
HROWS = N_PAD // CHUNK
NVREG = ER_PAD // NS // 16

TC_BLK = 2048


_sc_mesh = plsc.VectorSubcoreMesh(core_axis_name="c", subcore_axis_name="s")


@functools.partial(
    pl.kernel,
    out_type=(
        jax.ShapeDtypeStruct((NUM_RELS, N_PAD, IN_FEAT), jnp.float32),
        jax.ShapeDtypeStruct((NUM_RELS, NS, N_PAD), jnp.float32),
    ),
    mesh=_sc_mesh,
    compiler_params=pltpu.CompilerParams(needs_layout_passes=False),
    scratch_types=[
        pltpu.VMEM((CHUNKS_PER_TILE, CHUNK), jnp.int32),
        pltpu.VMEM((CHUNKS_PER_TILE, CHUNK), jnp.int32),
        pltpu.VMEM((CHUNK, IN_FEAT), jnp.float32),
        pltpu.VMEM((CHUNK, IN_FEAT), jnp.float32),
        pltpu.VMEM((N_PAD,), jnp.float32),
        pltpu.VMEM_SHARED((N_PAD, IN_FEAT), jnp.float32),
        pltpu.SemaphoreType.DMA,
        pltpu.SemaphoreType.DMA,
        pltpu.SemaphoreType.DMA,
    ],
)
def _sc_segment_sums(x_hbm, src_hbm, dst_hbm, z128_hbm, z1d_hbm,
                     agg_out, deg_out,
                     src_t, dst_t, rows_a, rows_b, deg_hist, acc,
                     gsem, gsem2, zsem):
    c = lax.axis_index("c")
    s = lax.axis_index("s")
    row0 = s * ROWS_PER_TILE

    def zero_own_rows():
        pltpu.sync_copy(z128_hbm, rows_a)
        cps = [pltpu.async_copy(rows_a, acc.at[pl.ds(row0 + j * CHUNK, CHUNK)],
                                zsem)
               for j in range(FLUSH_CHUNKS)]
        for cp in cps:
            cp.wait()

    def flush_own_rows(out_ref, r):
        cps = [None, None]
        for j in range(FLUSH_CHUNKS):
            b = j & 1
            buf = rows_a if b == 0 else rows_b
            sem = gsem if b == 0 else gsem2
            rows = pl.ds(row0 + j * CHUNK, CHUNK)
            if cps[b] is not None:
                cps[b].wait()
            pltpu.sync_copy(acc.at[rows], buf)
            cps[b] = pltpu.async_copy(buf, out_ref.at[r, rows], sem)
        for cp in cps:
            if cp is not None:
                cp.wait()

    iota16 = lax.iota(jnp.int32, 16)
    ones16 = jnp.ones((16,), jnp.float32)

    def hist_body(t, carry):
        v = dst_t[lax.shift_right_logical(t, 3),
                  pl.ds(jnp.bitwise_and(t, 7) * 16, 16)]
        plsc.addupdate_scatter(deg_hist, [v], ones16)
        return carry

    for rr in range(RELS_PER_CORE):
        r = c * RELS_PER_CORE + rr
        zero_own_rows()
        pltpu.sync_copy(z1d_hbm, deg_hist)
        plsc.subcore_barrier()

        pltpu.sync_copy(src_hbm.at[r, s], src_t)
        pltpu.sync_copy(dst_hbm.at[r, s], dst_t)
        cp = pltpu.async_copy(x_hbm.at[src_t.at[0]], rows_a, gsem)
        for k in range(CHUNKS_PER_TILE):
            buf = rows_a if k % 2 == 0 else rows_b
            nxt = rows_b if k % 2 == 0 else rows_a
            cp.wait()
            if k + 1 < CHUNKS_PER_TILE:
                cp = pltpu.async_copy(x_hbm.at[src_t.at[k + 1]], nxt, gsem)
            pltpu.sync_copy(buf, acc.at[dst_t.at[k]], add=True)

        lax.fori_loop(0, NVREG, hist_body, 0)
        pltpu.sync_copy(deg_hist, deg_out.at[r, s])

        plsc.subcore_barrier()
        flush_own_rows(agg_out, r)


def _tc_body(wc_ref, bases_ref, bias_ref, agg_ref, deg_ref, out_ref):
    acc = jnp.zeros((TC_BLK, OUT_FEAT), jnp.float32)
    for r in range(NUM_RELS):
        w_r = jnp.zeros((IN_FEAT, OUT_FEAT), jnp.float32)
        for b in range(NUM_BASES):
            w_r = w_r + wc_ref[r, b] * bases_ref[b]
        dsum = jnp.sum(deg_ref[r], axis=0, keepdims=True)
        d = jnp.maximum(jnp.transpose(dsum), 1.0)
        a = agg_ref[r] / d
        acc = acc + jnp.dot(a, w_r, preferred_element_type=jnp.float32,
                            precision=lax.Precision.HIGHEST)
    out_ref[...] = acc + bias_ref[...]


def _tc_combine(w_comp, bases, h_bias2d, agg, deg):
    return pl.pallas_call(
        _tc_body,
        grid=(N_PAD // TC_BLK,),
        in_specs=[
            pl.BlockSpec((NUM_RELS, NUM_BASES), lambda i: (0, 0),
                         memory_space=pltpu.SMEM),
            pl.BlockSpec((NUM_BASES, IN_FEAT, OUT_FEAT), lambda i: (0, 0, 0)),
            pl.BlockSpec((1, OUT_FEAT), lambda i: (0, 0)),
            pl.BlockSpec((NUM_RELS, TC_BLK, IN_FEAT), lambda i: (0, i, 0)),
            pl.BlockSpec((NUM_RELS, NS, TC_BLK), lambda i: (0, 0, i)),
        ],
        out_specs=pl.BlockSpec((TC_BLK, OUT_FEAT), lambda i: (i, 0)),
        out_shape=jax.ShapeDtypeStruct((N_PAD, OUT_FEAT), jnp.float32),
    )(w_comp, bases, h_bias2d, agg, deg)


def kernel(x, edge_index, w_comp, bases, h_bias):
    src = edge_index[0].astype(jnp.int32).reshape(NUM_RELS, ER)
    dst = edge_index[1].astype(jnp.int32).reshape(NUM_RELS, ER)
    pad = ER_PAD - ER
    src_p = jnp.pad(src, ((0, 0), (0, pad))).reshape(
        NUM_RELS, NS, CHUNKS_PER_TILE, CHUNK)
    dst_p = jnp.pad(dst, ((0, 0), (0, pad)), constant_values=N_NODES).reshape(
        NUM_RELS, NS, CHUNKS_PER_TILE, CHUNK)
    z128 = jnp.zeros((CHUNK, IN_FEAT), jnp.float32)
    z1d = jnp.zeros((N_PAD,), jnp.float32)
    agg, deg = _sc_segment_sums(x, src_p, dst_p, z128, z1d)
    out = _tc_combine(w_comp, bases, h_bias.reshape(1, OUT_FEAT), agg, deg)
    return out[:N_NODES]


# speedup vs baseline: 3.5349x; 1.0111x over previous
import functools

import jax
import jax.numpy as jnp
from jax import lax
from jax.experimental import pallas as pl
from jax.experimental.pallas import tpu as pltpu
from jax.experimental.pallas import tpu_sc as plsc

N_NODES = 10000
IN_FEAT = 128
OUT_FEAT = 128
NUM_RELS = 8
NUM_BASES = 4
N_EDGES = 320000
ER = N_EDGES // NUM_RELS

NC = 2
NS = 16
CHUNK = 128
CHUNKS_PER_TILE = 20
ER_PAD = NS * CHUNKS_PER_TILE * CHUNK
RELS_PER_CORE = NUM_RELS // NC

N_PAD = 10240
ROWS_PER_TILE = N_PAD // NS
FLUSH_CHUNKS = ROWS_PER_TILE // CHUNK
HROWS = N_PAD // CHUNK
NVREG = ER_PAD // NS // 16

TC_BLK = 2048


_sc_mesh = plsc.VectorSubcoreMesh(core_axis_name="c", subcore_axis_name="s")


@functools.partial(
    pl.kernel,
    out_type=(
        jax.ShapeDtypeStruct((NUM_RELS, N_PAD, IN_FEAT), jnp.float32),
        jax.ShapeDtypeStruct((NUM_RELS, NS, N_PAD), jnp.float32),
    ),
    mesh=_sc_mesh,
    compiler_params=pltpu.CompilerParams(needs_layout_passes=False),
    scratch_types=[
        pltpu.VMEM((CHUNKS_PER_TILE, CHUNK), jnp.int32),
        pltpu.VMEM((CHUNKS_PER_TILE, CHUNK), jnp.int32),
        pltpu.VMEM((CHUNK, IN_FEAT), jnp.float32),
        pltpu.VMEM((CHUNK, IN_FEAT), jnp.float32),
        pltpu.VMEM((N_PAD,), jnp.float32),
        pltpu.VMEM_SHARED((N_PAD, IN_FEAT), jnp.float32),
        pltpu.SemaphoreType.DMA,
        pltpu.SemaphoreType.DMA,
        pltpu.SemaphoreType.DMA,
        pltpu.SemaphoreType.DMA,
        pltpu.SemaphoreType.DMA,
        pltpu.SemaphoreType.DMA,
    ],
)
def _sc_segment_sums(x_hbm, src_hbm, dst_hbm, z128_hbm, z1d_hbm,
                     agg_out, deg_out,
                     src_t, dst_t, rows_a, rows_b, deg_hist, acc,
                     gsem, gsem2, zsem, psem1, psem2, hsem):
    c = lax.axis_index("c")
    s = lax.axis_index("s")
    row0 = s * ROWS_PER_TILE

    def zero_own_rows():
        pltpu.sync_copy(z128_hbm, rows_a)
        cps = [pltpu.async_copy(rows_a, acc.at[pl.ds(row0 + j * CHUNK, CHUNK)],
                                zsem)
               for j in range(FLUSH_CHUNKS)]
        for cp in cps:
            cp.wait()

    def flush_own_rows(out_ref, r):
        cps = [None, None]
        for j in range(FLUSH_CHUNKS):
            b = j & 1
            buf = rows_a if b == 0 else rows_b
            sem = gsem if b == 0 else gsem2
            rows = pl.ds(row0 + j * CHUNK, CHUNK)
            if cps[b] is not None:
                cps[b].wait()
            pltpu.sync_copy(acc.at[rows], buf)
            cps[b] = pltpu.async_copy(buf, out_ref.at[r, rows], sem)
        for cp in cps:
            if cp is not None:
                cp.wait()

    iota16 = lax.iota(jnp.int32, 16)
    ones16 = jnp.ones((16,), jnp.float32)

    def hist_body(t, carry):
        v = dst_t[lax.shift_right_logical(t, 3),
                  pl.ds(jnp.bitwise_and(t, 7) * 16, 16)]
        plsc.addupdate_scatter(deg_hist, [v], ones16)
        return carry

    src_cp = pltpu.async_copy(src_hbm.at[c * RELS_PER_CORE, s], src_t, psem1)
    dst_cp = pltpu.async_copy(dst_hbm.at[c * RELS_PER_CORE, s], dst_t, psem2)
    hist_cp = None

    for rr in range(RELS_PER_CORE):
        r = c * RELS_PER_CORE + rr
        zero_own_rows()
        if hist_cp is not None:
            hist_cp.wait()
        pltpu.sync_copy(z1d_hbm, deg_hist)
        plsc.subcore_barrier()

        src_cp.wait()
        dst_cp.wait()
        cp = pltpu.async_copy(x_hbm.at[src_t.at[0]], rows_a, gsem)
        for k in range(CHUNKS_PER_TILE):
            buf = rows_a if k % 2 == 0 else rows_b
            nxt = rows_b if k % 2 == 0 else rows_a
            cp.wait()
            if k + 1 < CHUNKS_PER_TILE:
                cp = pltpu.async_copy(x_hbm.at[src_t.at[k + 1]], nxt, gsem)
            pltpu.sync_copy(buf, acc.at[dst_t.at[k]], add=True)

        if rr + 1 < RELS_PER_CORE:
            src_cp = pltpu.async_copy(src_hbm.at[r + 1, s], src_t, psem1)

        lax.fori_loop(0, NVREG, hist_body, 0)
        hist_cp = pltpu.async_copy(deg_hist, deg_out.at[r, s], hsem)
        if rr + 1 < RELS_PER_CORE:
            dst_cp = pltpu.async_copy(dst_hbm.at[r + 1, s], dst_t, psem2)

        plsc.subcore_barrier()
        flush_own_rows(agg_out, r)

    hist_cp.wait()


def _tc_body(wc_ref, bases_ref, bias_ref, agg_ref, deg_ref, out_ref):
    acc = jnp.zeros((TC_BLK, OUT_FEAT), jnp.float32)
    for r in range(NUM_RELS):
        w_r = jnp.zeros((IN_FEAT, OUT_FEAT), jnp.float32)
        for b in range(NUM_BASES):
            w_r = w_r + wc_ref[r, b] * bases_ref[b]
        dsum = jnp.sum(deg_ref[r], axis=0, keepdims=True)
        d = jnp.maximum(jnp.transpose(dsum), 1.0)
        a = agg_ref[r] / d
        acc = acc + jnp.dot(a, w_r, preferred_element_type=jnp.float32,
                            precision=lax.Precision.HIGHEST)
    out_ref[...] = acc + bias_ref[...]


def _tc_combine(w_comp, bases, h_bias2d, agg, deg):
    return pl.pallas_call(
        _tc_body,
        grid=(N_PAD // TC_BLK,),
        in_specs=[
            pl.BlockSpec((NUM_RELS, NUM_BASES), lambda i: (0, 0),
                         memory_space=pltpu.SMEM),
            pl.BlockSpec((NUM_BASES, IN_FEAT, OUT_FEAT), lambda i: (0, 0, 0)),
            pl.BlockSpec((1, OUT_FEAT), lambda i: (0, 0)),
            pl.BlockSpec((NUM_RELS, TC_BLK, IN_FEAT), lambda i: (0, i, 0)),
            pl.BlockSpec((NUM_RELS, NS, TC_BLK), lambda i: (0, 0, i)),
        ],
        out_specs=pl.BlockSpec((TC_BLK, OUT_FEAT), lambda i: (i, 0)),
        out_shape=jax.ShapeDtypeStruct((N_PAD, OUT_FEAT), jnp.float32),
    )(w_comp, bases, h_bias2d, agg, deg)


def kernel(x, edge_index, w_comp, bases, h_bias):
    src = edge_index[0].astype(jnp.int32).reshape(NUM_RELS, ER)
    dst = edge_index[1].astype(jnp.int32).reshape(NUM_RELS, ER)
    pad = ER_PAD - ER
    src_p = jnp.pad(src, ((0, 0), (0, pad))).reshape(
        NUM_RELS, NS, CHUNKS_PER_TILE, CHUNK)
    dst_p = jnp.pad(dst, ((0, 0), (0, pad)), constant_values=N_NODES).reshape(
        NUM_RELS, NS, CHUNKS_PER_TILE, CHUNK)
    z128 = jnp.zeros((CHUNK, IN_FEAT), jnp.float32)
    z1d = jnp.zeros((N_PAD,), jnp.float32)
    agg, deg = _sc_segment_sums(x, src_p, dst_p, z128, z1d)
    out = _tc_combine(w_comp, bases, h_bias.reshape(1, OUT_FEAT), agg, deg)
    return out[:N_NODES]
